# baseline (speedup 1.0000x reference)
import functools
import math

import jax
import jax.numpy as jnp
from jax import lax
from jax.experimental import pallas as pl
from jax.experimental.pallas import tpu as pltpu
from jax.experimental.pallas import tpu_sc as plsc

_LANES = 16


def kernel(x, lut):
    orig_shape = x.shape
    B = x.size
    V, D = lut.shape
    info = plsc.get_sparse_core_info()
    NC, NS = info.num_cores, info.num_subcores
    NW = NC * NS
    CH = 128
    per_w = B // NW
    assert B % NW == 0 and per_w % CH == 0 and D % _LANES == 0
    n_chunks = per_w // CH
    scale = math.sqrt(D)

    x_flat = x.reshape(NW, n_chunks, CH).astype(jnp.int32)
    mesh = plsc.VectorSubcoreMesh(core_axis_name="c", subcore_axis_name="s")

    @functools.partial(
        pl.kernel,
        mesh=mesh,
        compiler_params=pltpu.CompilerParams(use_tc_tiling_on_sc=False),
        out_type=jax.ShapeDtypeStruct((B, D), jnp.float32),
        scratch_types=[
            pltpu.VMEM((n_chunks, CH), jnp.int32),
            pltpu.VMEM((CH, D), jnp.float32),
            pltpu.VMEM((CH, D), jnp.float32),
            pltpu.SemaphoreType.DMA,
        ],
    )
    def run(x_hbm, lut_hbm, out_hbm, idx_v, gbuf, sbuf, gsem):
        wid = lax.axis_index("s") * NC + lax.axis_index("c")
        pltpu.sync_copy(x_hbm.at[wid], idx_v)

        def chunk(c, carry):
            pltpu.async_copy(lut_hbm.at[idx_v.at[c]], gbuf, gsem).wait()

            def row(r, carry2):
                for j in range(D // _LANES):
                    sl = pl.ds(_LANES * j, _LANES)
                    sbuf[r, sl] = gbuf[r, sl] * scale
                return carry2

            lax.fori_loop(0, CH, row, 0)
            pltpu.sync_copy(sbuf, out_hbm.at[pl.ds(wid * per_w + c * CH, CH)])
            return carry

        lax.fori_loop(0, n_chunks, chunk, 0)

    out = run(x_flat, lut)
    return out.reshape(orig_shape + (D,))


# speedup vs baseline: 1.1817x; 1.1817x over previous
import functools
import math

import jax
import jax.numpy as jnp
from jax import lax
from jax.experimental import pallas as pl
from jax.experimental.pallas import tpu as pltpu
from jax.experimental.pallas import tpu_sc as plsc

_LANES = 16
_NBUF = 4
_RUNROLL = 4


def kernel(x, lut):
    orig_shape = x.shape
    B = x.size
    V, D = lut.shape
    info = plsc.get_sparse_core_info()
    NC, NS = info.num_cores, info.num_subcores
    NW = NC * NS
    CH = 128
    per_w = B // NW
    assert B % NW == 0 and per_w % (CH * _NBUF) == 0 and D % _LANES == 0
    n_chunks = per_w // CH
    scale = math.sqrt(D)

    x_flat = x.reshape(NW, n_chunks, CH).astype(jnp.int32)
    mesh = plsc.VectorSubcoreMesh(core_axis_name="c", subcore_axis_name="s")

    @functools.partial(
        pl.kernel,
        mesh=mesh,
        compiler_params=pltpu.CompilerParams(use_tc_tiling_on_sc=False),
        out_type=jax.ShapeDtypeStruct((B, D), jnp.float32),
        scratch_types=[
            pltpu.VMEM((n_chunks, CH), jnp.int32),
            [pltpu.VMEM((CH, D), jnp.float32)] * _NBUF,
            [pltpu.VMEM((CH, D), jnp.float32)] * _NBUF,
            [pltpu.SemaphoreType.DMA] * _NBUF,
            [pltpu.SemaphoreType.DMA] * _NBUF,
        ],
    )
    def run(x_hbm, lut_hbm, out_hbm, idx_v, gbufs, sbufs, gsems, ssems):
        wid = lax.axis_index("s") * NC + lax.axis_index("c")
        base = wid * per_w
        pltpu.sync_copy(x_hbm.at[wid], idx_v)

        for b in range(_NBUF):
            pltpu.async_copy(lut_hbm.at[idx_v.at[b]], gbufs[b], gsems[b])

        def step(it, carry):
            for b in range(_NBUF):
                c = it * _NBUF + b
                gbuf, sbuf = gbufs[b], sbufs[b]
                pltpu.make_async_copy(lut_hbm.at[idx_v.at[c]], gbuf,
                                      gsems[b]).wait()
                @pl.when(it > 0)
                def _():
                    pltpu.make_async_copy(
                        sbuf, out_hbm.at[pl.ds(0, CH)], ssems[b]).wait()

                def rows(r0, carry2):
                    r = r0 * _RUNROLL
                    for rr in range(_RUNROLL):
                        for j in range(D // _LANES):
                            sl = pl.ds(_LANES * j, _LANES)
                            sbuf[r + rr, sl] = gbuf[r + rr, sl] * scale
                    return carry2

                lax.fori_loop(0, CH // _RUNROLL, rows, 0)
                pltpu.async_copy(sbuf, out_hbm.at[pl.ds(base + c * CH, CH)],
                                 ssems[b])
                p = c + _NBUF

                @pl.when(p < n_chunks)
                def _():
                    pltpu.async_copy(lut_hbm.at[idx_v.at[p]], gbuf, gsems[b])

            return carry

        lax.fori_loop(0, n_chunks // _NBUF, step, 0)
        for b in range(_NBUF):
            pltpu.make_async_copy(sbufs[b], out_hbm.at[pl.ds(0, CH)],
                                  ssems[b]).wait()

    out = run(x_flat, lut)
    return out.reshape(orig_shape + (D,))
